# baseline (speedup 1.0000x reference)
import functools

import jax
import jax.numpy as jnp
from jax import lax
from jax.experimental import pallas as pl
from jax.experimental.pallas import tpu as pltpu
from jax.experimental.pallas import tpu_sc as plsc

N_NODES = 10000
N_EDGES = 320000
D_NODE = 128

NC = 2
NS = 16
NW = NC * NS
EW = N_EDGES // NW
CH = 1000
NCHUNK = EW // CH
ROWS_PER_TILE = N_NODES // NS

E4 = 4
E4R = N_EDGES // E4
RCH = CH // E4

_MESH = dict(core_axis_name="c", subcore_axis_name="s")
_SC_PARAMS = pltpu.CompilerParams(use_tc_tiling_on_sc=False)


def _worker(c, s):
    return s * NC + c


def _gather_kernel(half):
    mesh = plsc.VectorSubcoreMesh(**_MESH)
    EWH = EW // 2
    off = half * (N_EDGES // 2)

    @functools.partial(
        pl.kernel,
        out_type=(
            jax.ShapeDtypeStruct((E4R // 2, 128), jnp.float32),
            jax.ShapeDtypeStruct((E4R // 2, 128), jnp.float32),
        ),
        mesh=mesh,
        compiler_params=_SC_PARAMS,
        scratch_types=[
            pltpu.VMEM((CH,), jnp.int32),
            pltpu.VMEM((CH,), jnp.int32),
            pltpu.VMEM((CH, 32), jnp.float32),
            pltpu.VMEM((CH, 32), jnp.float32),
            pltpu.SemaphoreType.DMA,
            pltpu.SemaphoreType.DMA,
        ],
    )
    def gather(xa_hbm, xb_hbm, src_hbm, dst_hbm, ga_hbm, gb_hbm,
               ia_v, ib_v, ra_v, rb_v, sem_a, sem_b):
        c = lax.axis_index("c")
        s = lax.axis_index("s")
        wid = _worker(c, s)

        def chunk(k, _):
            base = off + wid * EWH + k * CH
            rbase = wid * (EWH // E4) + k * RCH
            pltpu.sync_copy(src_hbm.at[pl.ds(base, CH)], ia_v)
            pltpu.sync_copy(dst_hbm.at[pl.ds(base, CH)], ib_v)
            cp_a = pltpu.async_copy(xa_hbm.at[ia_v], ra_v, sem_a)
            cp_b = pltpu.async_copy(xb_hbm.at[ib_v], rb_v, sem_b)
            cp_a.wait()
            cp_b.wait()
            for j in range(E4):
                pltpu.sync_copy(ra_v.at[pl.ds(RCH * j, RCH)],
                                ga_hbm.at[pl.ds(rbase, RCH), pl.ds(32 * j, 32)])
                pltpu.sync_copy(rb_v.at[pl.ds(RCH * j, RCH)],
                                gb_hbm.at[pl.ds(rbase, RCH), pl.ds(32 * j, 32)])
            return 0
        lax.fori_loop(0, NCHUNK // 2, chunk, 0)

    return gather


def _scatter_kernel(with_deg):
    mesh = plsc.VectorSubcoreMesh(**_MESH)
    out_type = [jax.ShapeDtypeStruct((NC, N_NODES, 32), jnp.float32)]
    scratch = [
        pltpu.VMEM((CH,), jnp.int32),
        pltpu.VMEM((CH, 32), jnp.float32),
        pltpu.VMEM((ROWS_PER_TILE, 32), jnp.float32),
        pltpu.VMEM_SHARED((N_NODES, 32), jnp.float32),
    ]
    if with_deg:
        out_type.append(jax.ShapeDtypeStruct((NC, N_NODES, 16), jnp.float32))
        scratch.append(pltpu.VMEM((CH, 16), jnp.float32))
        scratch.append(pltpu.VMEM((ROWS_PER_TILE, 16), jnp.float32))
        scratch.append(pltpu.VMEM_SHARED((N_NODES, 16), jnp.float32))

    @functools.partial(
        pl.kernel,
        out_type=tuple(out_type),
        mesh=mesh,
        compiler_params=_SC_PARAMS,
        scratch_types=scratch,
    )
    def scatter(e0_hbm, e1_hbm, dst_hbm, *refs):
        if with_deg:
            (out_hbm, deg_hbm, idx_v, rows_v, buf_v, acc_sh,
             ones_v, dbuf_v, deg_sh) = refs
        else:
            out_hbm, idx_v, rows_v, buf_v, acc_sh = refs
        c = lax.axis_index("c")
        s = lax.axis_index("s")
        wid = _worker(c, s)
        zeros16 = jnp.zeros((16,), jnp.float32)
        ones16 = jnp.ones((16,), jnp.float32)

        def init_row(i, _):
            buf_v[i, pl.ds(0, 16)] = zeros16
            buf_v[i, pl.ds(16, 16)] = zeros16
            if with_deg:
                ones_v[i, :] = ones16
                dbuf_v[i, :] = zeros16
            return 0
        lax.fori_loop(0, ROWS_PER_TILE, init_row, 0)
        if with_deg:
            def ones_row(i, _):
                ones_v[i, :] = ones16
                return 0
            lax.fori_loop(ROWS_PER_TILE, CH, ones_row, 0)

        row0 = s * ROWS_PER_TILE
        pltpu.sync_copy(buf_v, acc_sh.at[pl.ds(row0, ROWS_PER_TILE)])
        if with_deg:
            pltpu.sync_copy(dbuf_v, deg_sh.at[pl.ds(row0, ROWS_PER_TILE)])
        plsc.subcore_barrier()

        def chunk(k, _):
            base = wid * EW + k * CH
            pltpu.sync_copy(dst_hbm.at[pl.ds(base, CH)], idx_v)
            rb0 = wid * (EW // E4) + k * RCH
            rb1 = rb0 - (E4R // 2)

            @pl.when(wid < NW // 2)
            def _():
                for j in range(E4):
                    pltpu.sync_copy(
                        e0_hbm.at[pl.ds(rb0, RCH), pl.ds(32 * j, 32)],
                        rows_v.at[pl.ds(RCH * j, RCH)])

            @pl.when(wid >= NW // 2)
            def _():
                for j in range(E4):
                    pltpu.sync_copy(
                        e1_hbm.at[pl.ds(rb1, RCH), pl.ds(32 * j, 32)],
                        rows_v.at[pl.ds(RCH * j, RCH)])

            pltpu.sync_copy(rows_v, acc_sh.at[idx_v], add=True)
            if with_deg:
                pltpu.sync_copy(ones_v, deg_sh.at[idx_v], add=True)
            return 0
        lax.fori_loop(0, NCHUNK, chunk, 0)
        plsc.subcore_barrier()

        pltpu.sync_copy(acc_sh.at[pl.ds(row0, ROWS_PER_TILE)], buf_v)
        pltpu.sync_copy(buf_v, out_hbm.at[c, pl.ds(row0, ROWS_PER_TILE)])
        if with_deg:
            pltpu.sync_copy(deg_sh.at[pl.ds(row0, ROWS_PER_TILE)], dbuf_v)
            pltpu.sync_copy(dbuf_v, deg_hbm.at[c, pl.ds(row0, ROWS_PER_TILE)])

    return scatter


BLK_E = 3200
GRID_E = N_EDGES // BLK_E
BLK_R = BLK_E // E4
BLK_N = 2000
GRID_N = N_NODES // BLK_N


def _proj_body(x_ref, w_ref, o_ref):
    o_ref[...] = jnp.dot(x_ref[...], w_ref[...],
                         preferred_element_type=jnp.float32)


def _proj(x, wab):
    return pl.pallas_call(
        _proj_body,
        grid=(GRID_N,),
        in_specs=[
            pl.BlockSpec((BLK_N, D_NODE), lambda i: (i, 0)),
            pl.BlockSpec((D_NODE, 64), lambda i: (0, 0)),
        ],
        out_specs=pl.BlockSpec((BLK_N, 64), lambda i: (i, 0)),
        out_shape=jax.ShapeDtypeStruct((N_NODES, 64), jnp.float32),
    )(x, wab)


def _edge_body(ga_ref, gb_ref, e_ref, wc_ref, be_ref, o_ref):
    t = jnp.dot(e_ref[...], wc_ref[...], preferred_element_type=jnp.float32)
    o_ref[...] = jnp.maximum(t + ga_ref[...] + gb_ref[...] + be_ref[...], 0.0)


def _edge_stage(ga, gb, e_prev, wc_bd, be_t):
    de4 = e_prev.shape[1]
    rows = ga.shape[0]
    e_spec = pl.BlockSpec((BLK_R, de4), lambda i: (i, 0))
    return pl.pallas_call(
        _edge_body,
        grid=(rows // BLK_R,),
        in_specs=[
            pl.BlockSpec((BLK_R, 128), lambda i: (i, 0)),
            pl.BlockSpec((BLK_R, 128), lambda i: (i, 0)),
            e_spec,
            pl.BlockSpec((de4, 128), lambda i: (0, 0)),
            pl.BlockSpec((1, 128), lambda i: (0, 0)),
        ],
        out_specs=pl.BlockSpec((BLK_R, 128), lambda i: (i, 0)),
        out_shape=jax.ShapeDtypeStruct((rows, 128), jnp.float32),
    )(ga, gb, e_prev, wc_bd, be_t)


def _node_body(mix, x_ref, aggp_ref, degp_ref, wna_ref, wnb_ref, bn_ref,
               wea_ref, web_ref, n_ref, xa_ref, xb_ref):
    deg = degp_ref[0, :, 0:1] + degp_ref[1, :, 0:1]
    inv = 1.0 / jnp.maximum(deg, 1.0)
    agg = (aggp_ref[0] + aggp_ref[1]) * inv
    x = x_ref[...]
    n = jnp.dot(x, wna_ref[...], preferred_element_type=jnp.float32)
    n = n + jnp.dot(agg, wnb_ref[...], preferred_element_type=jnp.float32)
    n = jnp.maximum(n + bn_ref[...], 0.0)
    n_ref[...] = n
    p = 0.5 * (x + n) if mix else n
    xa_ref[...] = jnp.dot(p, wea_ref[...], preferred_element_type=jnp.float32)
    xb_ref[...] = jnp.dot(p, web_ref[...], preferred_element_type=jnp.float32)


def _node_stage(x, aggp, degp, wna, wnb, bn, wea, web, mix):
    return pl.pallas_call(
        functools.partial(_node_body, mix),
        grid=(GRID_N,),
        in_specs=[
            pl.BlockSpec((BLK_N, D_NODE), lambda i: (i, 0)),
            pl.BlockSpec((NC, BLK_N, 32), lambda i: (0, i, 0)),
            pl.BlockSpec((NC, BLK_N, 16), lambda i: (0, i, 0)),
            pl.BlockSpec((D_NODE, D_NODE), lambda i: (0, 0)),
            pl.BlockSpec((32, D_NODE), lambda i: (0, 0)),
            pl.BlockSpec((1, D_NODE), lambda i: (0, 0)),
            pl.BlockSpec((D_NODE, 32), lambda i: (0, 0)),
            pl.BlockSpec((D_NODE, 32), lambda i: (0, 0)),
        ],
        out_specs=[
            pl.BlockSpec((BLK_N, D_NODE), lambda i: (i, 0)),
            pl.BlockSpec((BLK_N, 32), lambda i: (i, 0)),
            pl.BlockSpec((BLK_N, 32), lambda i: (i, 0)),
        ],
        out_shape=[
            jax.ShapeDtypeStruct((N_NODES, D_NODE), jnp.float32),
            jax.ShapeDtypeStruct((N_NODES, 32), jnp.float32),
            jax.ShapeDtypeStruct((N_NODES, 32), jnp.float32),
        ],
    )(x, aggp, degp, wna, wnb, bn, wea, web)


def _final_body(ga_ref, gb_ref, e0_ref, e1_ref, wc_ref, be_ref,
                wm1_ref, bm1_ref, wm2_ref, bm2_ref, wm3_ref, bm3_ref, o_ref):
    em = 0.5 * (e0_ref[...] + e1_ref[...])
    t = jnp.dot(em, wc_ref[...], preferred_element_type=jnp.float32)
    e2 = jnp.maximum(t + ga_ref[...] + gb_ref[...] + be_ref[...], 0.0)
    h = jnp.maximum(jnp.dot(e2, wm1_ref[...],
                            preferred_element_type=jnp.float32)
                    + bm1_ref[...], 0.0)
    h = jnp.maximum(jnp.dot(h, wm2_ref[...],
                            preferred_element_type=jnp.float32)
                    + bm2_ref[...], 0.0)
    o_ref[...] = (jnp.dot(h, wm3_ref[...], preferred_element_type=jnp.float32)
                  + bm3_ref[0]).reshape(1, BLK_R, E4)


def _final_stage(ga, gb, e0, e1, wc_bd, be_t, wm1_bd, bm1_t, wm2_bd, bm2_t,
                 wm3_bd, bm3):
    bd = lambda: pl.BlockSpec((128, 128), lambda i: (0, 0))
    row = lambda: pl.BlockSpec((1, 128), lambda i: (0, 0))
    ebs = lambda: pl.BlockSpec((BLK_R, 128), lambda i: (i, 0))
    grid = ga.shape[0] // BLK_R
    return pl.pallas_call(
        _final_body,
        grid=(grid,),
        in_specs=[
            ebs(), ebs(), ebs(), ebs(),
            bd(), row(),
            bd(), row(),
            bd(), row(),
            pl.BlockSpec((128, E4), lambda i: (0, 0)),
            pl.BlockSpec(memory_space=pltpu.SMEM),
        ],
        out_specs=pl.BlockSpec((1, BLK_R, E4), lambda i: (i, 0, 0)),
        out_shape=jax.ShapeDtypeStruct((grid, BLK_R, E4), jnp.float32),
    )(ga, gb, e0, e1, wc_bd, be_t, wm1_bd, bm1_t, wm2_bd, bm2_t, wm3_bd, bm3)


def kernel(edge_index, x, edge_attr,
           We0, be0, Wn0, bn0,
           We1, be1, Wn1, bn1,
           We2, be2, Wn2, bn2,
           Wm1, bm1, Wm2, bm2, Wm3, bm3):
    x = x.astype(jnp.float32)
    perm = lambda a: (a.reshape(NW * NCHUNK, RCH, E4)
                      .transpose(0, 2, 1).reshape(N_EDGES))
    src = perm(edge_index[0])
    dst = perm(edge_index[1])

    gather_h0 = _gather_kernel(0)
    gather_h1 = _gather_kernel(1)
    scatter0 = _scatter_kernel(with_deg=True)
    scatter1 = _scatter_kernel(with_deg=False)

    wa0, wb0, wc0 = We0[:128], We0[128:256], We0[256:]
    wa1, wb1, wc1 = We1[:128], We1[128:256], We1[256:]
    wa2, wb2, wc2 = We2[:128], We2[128:256], We2[256:]
    wn0a, wn0b = Wn0[:128], Wn0[128:]
    wn1a, wn1b = Wn1[:128], Wn1[128:]
    bn0r, bn1r = bn0.reshape(1, D_NODE), bn1.reshape(1, D_NODE)
    eye4 = jnp.eye(E4, dtype=jnp.float32)
    wc0_bd = jnp.kron(eye4, wc0)
    wc1_bd = jnp.kron(eye4, wc1)
    wc2_bd = jnp.kron(eye4, wc2)
    wm1_bd = jnp.kron(eye4, Wm1)
    wm2_bd = jnp.kron(eye4, Wm2)
    wm3_bd = jnp.kron(eye4, Wm3)
    be0_t = jnp.tile(be0, E4).reshape(1, 128)
    be1_t = jnp.tile(be1, E4).reshape(1, 128)
    be2_t = jnp.tile(be2, E4).reshape(1, 128)
    bm1_t = jnp.tile(bm1, E4).reshape(1, 128)
    bm2_t = jnp.tile(bm2, E4).reshape(1, 128)

    ea4 = edge_attr.reshape(2, E4R // 2, E4 * 16)

    def layer_edges(xa, xb, e_prev_halves, wc_bd, be_t):
        ga0h, gb0h = gather_h0(xa, xb, src, dst)
        ga1h, gb1h = gather_h1(xa, xb, src, dst)
        eh0 = _edge_stage(ga0h, gb0h, e_prev_halves[0], wc_bd, be_t)
        eh1 = _edge_stage(ga1h, gb1h, e_prev_halves[1], wc_bd, be_t)
        return eh0, eh1

    xab0 = _proj(x, jnp.concatenate([wa0, wb0], axis=1))
    e0h = layer_edges(xab0[:, :32], xab0[:, 32:], (ea4[0], ea4[1]),
                      wc0_bd, be0_t)
    aggp0, degp = scatter0(e0h[0], e0h[1], dst)
    n0, xa1, xb1 = _node_stage(x, aggp0, degp, wn0a, wn0b, bn0r,
                               wa1, wb1, mix=False)

    e1h = layer_edges(xa1, xb1, e0h, wc1_bd, be1_t)
    (aggp1,) = scatter1(e1h[0], e1h[1], dst)
    _, xa2, xb2 = _node_stage(n0, aggp1, degp, wn1a, wn1b, bn1r,
                              wa2, wb2, mix=True)

    ga2h0, gb2h0 = gather_h0(xa2, xb2, src, dst)
    ga2h1, gb2h1 = gather_h1(xa2, xb2, src, dst)
    mlp_w = (wc2_bd, be2_t, wm1_bd, bm1_t, wm2_bd, bm2_t, wm3_bd, bm3)
    out0 = _final_stage(ga2h0, gb2h0, e0h[0], e1h[0], *mlp_w)
    out1 = _final_stage(ga2h1, gb2h1, e0h[1], e1h[1], *mlp_w)
    return jnp.concatenate([out0, out1]).reshape(N_EDGES)


# speedup vs baseline: 1.0706x; 1.0706x over previous
import functools

import jax
import jax.numpy as jnp
from jax import lax
from jax.experimental import pallas as pl
from jax.experimental.pallas import tpu as pltpu
from jax.experimental.pallas import tpu_sc as plsc

N_NODES = 10000
N_EDGES = 320000
D_NODE = 128

NC = 2
NS = 16
NW = NC * NS
EW = N_EDGES // NW
CH = 1000
NCHUNK = EW // CH
ROWS_PER_TILE = N_NODES // NS

E4 = 4
E4R = N_EDGES // E4
RCH = CH // E4

_MESH = dict(core_axis_name="c", subcore_axis_name="s")
_SC_PARAMS = pltpu.CompilerParams(use_tc_tiling_on_sc=False)


def _worker(c, s):
    return s * NC + c


def _gather_kernel():
    mesh = plsc.VectorSubcoreMesh(**_MESH)

    @functools.partial(
        pl.kernel,
        out_type=(
            jax.ShapeDtypeStruct((E4R, 128), jnp.float32),
            jax.ShapeDtypeStruct((E4R, 128), jnp.float32),
        ),
        mesh=mesh,
        compiler_params=_SC_PARAMS,
        scratch_types=[
            pltpu.VMEM((CH,), jnp.int32),
            pltpu.VMEM((CH,), jnp.int32),
            pltpu.VMEM((CH, 32), jnp.float32),
            pltpu.VMEM((CH, 32), jnp.float32),
            pltpu.SemaphoreType.DMA,
            pltpu.SemaphoreType.DMA,
        ],
    )
    def gather(xa_hbm, xb_hbm, src_hbm, dst_hbm, ga_hbm, gb_hbm,
               ia_v, ib_v, ra_v, rb_v, sem_a, sem_b):
        c = lax.axis_index("c")
        s = lax.axis_index("s")
        wid = _worker(c, s)

        def chunk(k, _):
            base = wid * EW + k * CH
            rbase = wid * (EW // E4) + k * RCH
            pltpu.sync_copy(src_hbm.at[pl.ds(base, CH)], ia_v)
            pltpu.sync_copy(dst_hbm.at[pl.ds(base, CH)], ib_v)
            cp_a = pltpu.async_copy(xa_hbm.at[ia_v], ra_v, sem_a)
            cp_b = pltpu.async_copy(xb_hbm.at[ib_v], rb_v, sem_b)
            cp_a.wait()
            cp_b.wait()
            for j in range(E4):
                pltpu.sync_copy(ra_v.at[pl.ds(RCH * j, RCH)],
                                ga_hbm.at[pl.ds(rbase, RCH), pl.ds(32 * j, 32)])
                pltpu.sync_copy(rb_v.at[pl.ds(RCH * j, RCH)],
                                gb_hbm.at[pl.ds(rbase, RCH), pl.ds(32 * j, 32)])
            return 0
        lax.fori_loop(0, NCHUNK, chunk, 0)

    return gather


def _scatter_kernel(with_deg):
    mesh = plsc.VectorSubcoreMesh(**_MESH)
    out_type = [jax.ShapeDtypeStruct((NC, N_NODES, 32), jnp.float32)]
    scratch = [
        pltpu.VMEM((CH,), jnp.int32),
        pltpu.VMEM((CH, 32), jnp.float32),
        pltpu.VMEM((ROWS_PER_TILE, 32), jnp.float32),
        pltpu.VMEM_SHARED((N_NODES, 32), jnp.float32),
    ]
    if with_deg:
        out_type.append(jax.ShapeDtypeStruct((NC, N_NODES, 16), jnp.float32))
        scratch.append(pltpu.VMEM((CH, 16), jnp.float32))
        scratch.append(pltpu.VMEM((ROWS_PER_TILE, 16), jnp.float32))
        scratch.append(pltpu.VMEM_SHARED((N_NODES, 16), jnp.float32))

    @functools.partial(
        pl.kernel,
        out_type=tuple(out_type),
        mesh=mesh,
        compiler_params=_SC_PARAMS,
        scratch_types=scratch,
    )
    def scatter(e_hbm, dst_hbm, *refs):
        if with_deg:
            (out_hbm, deg_hbm, idx_v, rows_v, buf_v, acc_sh,
             ones_v, dbuf_v, deg_sh) = refs
        else:
            out_hbm, idx_v, rows_v, buf_v, acc_sh = refs
        c = lax.axis_index("c")
        s = lax.axis_index("s")
        wid = _worker(c, s)
        zeros16 = jnp.zeros((16,), jnp.float32)
        ones16 = jnp.ones((16,), jnp.float32)

        def init_row(i, _):
            buf_v[i, pl.ds(0, 16)] = zeros16
            buf_v[i, pl.ds(16, 16)] = zeros16
            if with_deg:
                ones_v[i, :] = ones16
                dbuf_v[i, :] = zeros16
            return 0
        lax.fori_loop(0, ROWS_PER_TILE, init_row, 0)
        if with_deg:
            def ones_row(i, _):
                ones_v[i, :] = ones16
                return 0
            lax.fori_loop(ROWS_PER_TILE, CH, ones_row, 0)

        row0 = s * ROWS_PER_TILE
        pltpu.sync_copy(buf_v, acc_sh.at[pl.ds(row0, ROWS_PER_TILE)])
        if with_deg:
            pltpu.sync_copy(dbuf_v, deg_sh.at[pl.ds(row0, ROWS_PER_TILE)])
        plsc.subcore_barrier()

        def chunk(k, _):
            base = wid * EW + k * CH
            rbase = wid * (EW // E4) + k * RCH
            pltpu.sync_copy(dst_hbm.at[pl.ds(base, CH)], idx_v)
            for j in range(E4):
                pltpu.sync_copy(e_hbm.at[pl.ds(rbase, RCH), pl.ds(32 * j, 32)],
                                rows_v.at[pl.ds(RCH * j, RCH)])
            pltpu.sync_copy(rows_v, acc_sh.at[idx_v], add=True)
            if with_deg:
                pltpu.sync_copy(ones_v, deg_sh.at[idx_v], add=True)
            return 0
        lax.fori_loop(0, NCHUNK, chunk, 0)
        plsc.subcore_barrier()

        pltpu.sync_copy(acc_sh.at[pl.ds(row0, ROWS_PER_TILE)], buf_v)
        pltpu.sync_copy(buf_v, out_hbm.at[c, pl.ds(row0, ROWS_PER_TILE)])
        if with_deg:
            pltpu.sync_copy(deg_sh.at[pl.ds(row0, ROWS_PER_TILE)], dbuf_v)
            pltpu.sync_copy(dbuf_v, deg_hbm.at[c, pl.ds(row0, ROWS_PER_TILE)])

    return scatter


BLK_E = 3200
GRID_E = N_EDGES // BLK_E
BLK_R = BLK_E // E4
BLK_N = 2000
GRID_N = N_NODES // BLK_N


def _proj_body(x_ref, w_ref, o_ref):
    o_ref[...] = jnp.dot(x_ref[...], w_ref[...],
                         preferred_element_type=jnp.float32)


def _proj(x, wab):
    return pl.pallas_call(
        _proj_body,
        grid=(GRID_N,),
        in_specs=[
            pl.BlockSpec((BLK_N, D_NODE), lambda i: (i, 0)),
            pl.BlockSpec((D_NODE, 64), lambda i: (0, 0)),
        ],
        out_specs=pl.BlockSpec((BLK_N, 64), lambda i: (i, 0)),
        out_shape=jax.ShapeDtypeStruct((N_NODES, 64), jnp.float32),
    )(x, wab)


def _edge_body(ga_ref, gb_ref, e_ref, wc_ref, be_ref, o_ref):
    t = jnp.dot(e_ref[...], wc_ref[...], preferred_element_type=jnp.float32)
    o_ref[...] = jnp.maximum(t + ga_ref[...] + gb_ref[...] + be_ref[...], 0.0)


def _edge_stage(ga, gb, e_prev, wc_bd, be_t):
    de4 = e_prev.shape[1]
    e_spec = pl.BlockSpec((BLK_R, de4), lambda i: (i, 0))
    return pl.pallas_call(
        _edge_body,
        grid=(GRID_E,),
        in_specs=[
            pl.BlockSpec((BLK_R, 128), lambda i: (i, 0)),
            pl.BlockSpec((BLK_R, 128), lambda i: (i, 0)),
            e_spec,
            pl.BlockSpec((de4, 128), lambda i: (0, 0)),
            pl.BlockSpec((1, 128), lambda i: (0, 0)),
        ],
        out_specs=pl.BlockSpec((BLK_R, 128), lambda i: (i, 0)),
        out_shape=jax.ShapeDtypeStruct((E4R, 128), jnp.float32),
    )(ga, gb, e_prev, wc_bd, be_t)


def _node_body(mix, x_ref, aggp_ref, degp_ref, wna_ref, wnb_ref, bn_ref,
               wea_ref, web_ref, n_ref, xa_ref, xb_ref):
    deg = degp_ref[0, :, 0:1] + degp_ref[1, :, 0:1]
    inv = 1.0 / jnp.maximum(deg, 1.0)
    agg = (aggp_ref[0] + aggp_ref[1]) * inv
    x = x_ref[...]
    n = jnp.dot(x, wna_ref[...], preferred_element_type=jnp.float32)
    n = n + jnp.dot(agg, wnb_ref[...], preferred_element_type=jnp.float32)
    n = jnp.maximum(n + bn_ref[...], 0.0)
    n_ref[...] = n
    p = 0.5 * (x + n) if mix else n
    xa_ref[...] = jnp.dot(p, wea_ref[...], preferred_element_type=jnp.float32)
    xb_ref[...] = jnp.dot(p, web_ref[...], preferred_element_type=jnp.float32)


def _node_stage(x, aggp, degp, wna, wnb, bn, wea, web, mix):
    return pl.pallas_call(
        functools.partial(_node_body, mix),
        grid=(GRID_N,),
        in_specs=[
            pl.BlockSpec((BLK_N, D_NODE), lambda i: (i, 0)),
            pl.BlockSpec((NC, BLK_N, 32), lambda i: (0, i, 0)),
            pl.BlockSpec((NC, BLK_N, 16), lambda i: (0, i, 0)),
            pl.BlockSpec((D_NODE, D_NODE), lambda i: (0, 0)),
            pl.BlockSpec((32, D_NODE), lambda i: (0, 0)),
            pl.BlockSpec((1, D_NODE), lambda i: (0, 0)),
            pl.BlockSpec((D_NODE, 32), lambda i: (0, 0)),
            pl.BlockSpec((D_NODE, 32), lambda i: (0, 0)),
        ],
        out_specs=[
            pl.BlockSpec((BLK_N, D_NODE), lambda i: (i, 0)),
            pl.BlockSpec((BLK_N, 32), lambda i: (i, 0)),
            pl.BlockSpec((BLK_N, 32), lambda i: (i, 0)),
        ],
        out_shape=[
            jax.ShapeDtypeStruct((N_NODES, D_NODE), jnp.float32),
            jax.ShapeDtypeStruct((N_NODES, 32), jnp.float32),
            jax.ShapeDtypeStruct((N_NODES, 32), jnp.float32),
        ],
    )(x, aggp, degp, wna, wnb, bn, wea, web)


def _final_body(ga_ref, gb_ref, e0_ref, e1_ref, wc_ref, be_ref,
                wm1_ref, bm1_ref, wm2_ref, bm2_ref, wm3_ref, bm3_ref, o_ref):
    em = 0.5 * (e0_ref[...] + e1_ref[...])
    t = jnp.dot(em, wc_ref[...], preferred_element_type=jnp.float32)
    e2 = jnp.maximum(t + ga_ref[...] + gb_ref[...] + be_ref[...], 0.0)
    h = jnp.maximum(jnp.dot(e2, wm1_ref[...],
                            preferred_element_type=jnp.float32)
                    + bm1_ref[...], 0.0)
    h = jnp.maximum(jnp.dot(h, wm2_ref[...],
                            preferred_element_type=jnp.float32)
                    + bm2_ref[...], 0.0)
    o_ref[...] = (jnp.dot(h, wm3_ref[...], preferred_element_type=jnp.float32)
                  + bm3_ref[0]).reshape(1, BLK_R, E4)


def _final_stage(ga, gb, e0, e1, wc_bd, be_t, wm1_bd, bm1_t, wm2_bd, bm2_t,
                 wm3_bd, bm3):
    bd = lambda: pl.BlockSpec((128, 128), lambda i: (0, 0))
    row = lambda: pl.BlockSpec((1, 128), lambda i: (0, 0))
    ebs = lambda: pl.BlockSpec((BLK_R, 128), lambda i: (i, 0))
    return pl.pallas_call(
        _final_body,
        grid=(GRID_E,),
        in_specs=[
            ebs(), ebs(), ebs(), ebs(),
            bd(), row(),
            bd(), row(),
            bd(), row(),
            pl.BlockSpec((128, E4), lambda i: (0, 0)),
            pl.BlockSpec(memory_space=pltpu.SMEM),
        ],
        out_specs=pl.BlockSpec((1, BLK_R, E4), lambda i: (i, 0, 0)),
        out_shape=jax.ShapeDtypeStruct((GRID_E, BLK_R, E4), jnp.float32),
    )(ga, gb, e0, e1, wc_bd, be_t, wm1_bd, bm1_t, wm2_bd, bm2_t, wm3_bd, bm3)


def kernel(edge_index, x, edge_attr,
           We0, be0, Wn0, bn0,
           We1, be1, Wn1, bn1,
           We2, be2, Wn2, bn2,
           Wm1, bm1, Wm2, bm2, Wm3, bm3):
    x = x.astype(jnp.float32)
    perm = lambda a: (a.reshape(NW * NCHUNK, RCH, E4)
                      .transpose(0, 2, 1).reshape(N_EDGES))
    src = perm(edge_index[0])
    dst = perm(edge_index[1])

    gather = _gather_kernel()
    scatter0 = _scatter_kernel(with_deg=True)
    scatter1 = _scatter_kernel(with_deg=False)

    wa0, wb0, wc0 = We0[:128], We0[128:256], We0[256:]
    wa1, wb1, wc1 = We1[:128], We1[128:256], We1[256:]
    wa2, wb2, wc2 = We2[:128], We2[128:256], We2[256:]
    wn0a, wn0b = Wn0[:128], Wn0[128:]
    wn1a, wn1b = Wn1[:128], Wn1[128:]
    bn0r, bn1r = bn0.reshape(1, D_NODE), bn1.reshape(1, D_NODE)
    eye4 = jnp.eye(E4, dtype=jnp.float32)
    wc0_bd = jnp.kron(eye4, wc0)
    wc1_bd = jnp.kron(eye4, wc1)
    wc2_bd = jnp.kron(eye4, wc2)
    wm1_bd = jnp.kron(eye4, Wm1)
    wm2_bd = jnp.kron(eye4, Wm2)
    wm3_bd = jnp.kron(eye4, Wm3)
    be0_t = jnp.tile(be0, E4).reshape(1, 128)
    be1_t = jnp.tile(be1, E4).reshape(1, 128)
    be2_t = jnp.tile(be2, E4).reshape(1, 128)
    bm1_t = jnp.tile(bm1, E4).reshape(1, 128)
    bm2_t = jnp.tile(bm2, E4).reshape(1, 128)

    xab0 = _proj(x, jnp.concatenate([wa0, wb0], axis=1))
    ga0, gb0 = gather(xab0[:, :32], xab0[:, 32:], src, dst)
    e0 = _edge_stage(ga0, gb0, edge_attr.reshape(E4R, E4 * 16), wc0_bd, be0_t)
    aggp0, degp = scatter0(e0, dst)
    n0, xa1, xb1 = _node_stage(x, aggp0, degp, wn0a, wn0b, bn0r,
                               wa1, wb1, mix=False)

    ga1, gb1 = gather(xa1, xb1, src, dst)
    e1 = _edge_stage(ga1, gb1, e0, wc1_bd, be1_t)
    (aggp1,) = scatter1(e1, dst)
    _, xa2, xb2 = _node_stage(n0, aggp1, degp, wn1a, wn1b, bn1r,
                              wa2, wb2, mix=True)

    ga2, gb2 = gather(xa2, xb2, src, dst)
    out = _final_stage(ga2, gb2, e0, e1,
                       wc2_bd, be2_t, wm1_bd, bm1_t, wm2_bd, bm2_t,
                       wm3_bd, bm3)
    return out.reshape(N_EDGES)
